# baseline (speedup 1.0000x reference)
import functools

import jax
import jax.numpy as jnp
from jax import lax
from jax.experimental import pallas as pl
from jax.experimental.pallas import tpu as pltpu
from jax.experimental.pallas import tpu_sc as plsc

NC = 2
NS = 16
L = 16
NW = NC * NS

CP = 16
P = 128
NG = P // L
ROWS = 8 * P
IDXW = 128
NCPY = ROWS // IDXW


def _tec_body(d0, d1, d2, c_out,
              grid_hbm, xs_hbm, ys_hbm, zs_hbm, coef_hbm, out_hbm,
              coef_v, xs_v, ys_v, zs_v, idx_v, w_v, vals_v, out_v, sem):
    core = lax.axis_index("c")
    sub = lax.axis_index("s")
    wid = sub * NC + core
    n = out_hbm.shape[0] // c_out
    per_w = n // NW
    nchunk = per_w // P

    pltpu.sync_copy(coef_hbm, coef_v)

    iota = lax.iota(jnp.int32, L)

    def chunk_body(ci, carry):
        base = wid * per_w + ci * P
        pltpu.sync_copy(xs_hbm.at[pl.ds(base, P)], xs_v)
        pltpu.sync_copy(ys_hbm.at[pl.ds(base, P)], ys_v)
        pltpu.sync_copy(zs_hbm.at[pl.ds(base, P)], zs_v)

        for g in range(NG):
            s = g * L
            px = xs_v[pl.ds(s, L)] * coef_v[pl.ds(0, L)] + coef_v[pl.ds(3 * L, L)]
            py = ys_v[pl.ds(s, L)] * coef_v[pl.ds(L, L)] + coef_v[pl.ds(4 * L, L)]
            pz = zs_v[pl.ds(s, L)] * coef_v[pl.ds(2 * L, L)] + coef_v[pl.ds(5 * L, L)]
            valid = ((px >= 0.0) & (px <= d0 - 1.0)
                     & (py >= 0.0) & (py <= d1 - 1.0)
                     & (pz >= 0.0) & (pz <= d2 - 1.0))
            validf = jnp.where(valid, 1.0, 0.0).astype(jnp.float32)
            px = jnp.clip(px, 0.0, d0 - 1.0)
            py = jnp.clip(py, 0.0, d1 - 1.0)
            pz = jnp.clip(pz, 0.0, d2 - 1.0)
            ix = jnp.minimum(px.astype(jnp.int32), d0 - 2)
            iy = jnp.minimum(py.astype(jnp.int32), d1 - 2)
            iz = jnp.minimum(pz.astype(jnp.int32), d2 - 2)
            fx = px - ix.astype(jnp.float32)
            fy = py - iy.astype(jnp.float32)
            fz = pz - iz.astype(jnp.float32)
            gxv = (1.0 - fx) * validf
            fxv = fx * validf
            gy = 1.0 - fy
            gz = 1.0 - fz
            a = gy * gz
            b = gy * fz
            cc = fy * gz
            dd = fy * fz
            w8 = (gxv * a, gxv * b, gxv * cc, gxv * dd,
                  fxv * a, fxv * b, fxv * cc, fxv * dd)
            rbase = ix * (d1 * d2) + iy * d2 + iz
            offs = (0, 1, d2, d2 + 1,
                    d1 * d2, d1 * d2 + 1, d1 * d2 + d2, d1 * d2 + d2 + 1)
            for k in range(8):
                idx_v[pl.ds(k * P + s, L)] = rbase + offs[k]
                w_v[pl.ds(k * P + s, L)] = w8[k]

        copies = []
        for j in range(NCPY):
            copies.append(pltpu.async_copy(
                grid_hbm.at[idx_v.at[pl.ds(j * IDXW, IDXW)]],
                vals_v.at[pl.ds(j * IDXW, IDXW)],
                sem))
        for cp in copies:
            cp.wait()

        for g in range(NG):
            s = g * L
            rows = [iota + (k * P + s) for k in range(8)]
            wk = [w_v[pl.ds(k * P + s, L)] for k in range(8)]
            oidx = (iota + s) * c_out
            for c in range(c_out):
                colv = jnp.full((L,), c, jnp.int32)
                acc = wk[0] * plsc.load_gather(vals_v, [rows[0], colv])
                for k in range(1, 8):
                    acc = acc + wk[k] * plsc.load_gather(vals_v, [rows[k], colv])
                plsc.store_scatter(out_v, [oidx + c], acc)

        pltpu.sync_copy(out_v, out_hbm.at[pl.ds(base * c_out, P * c_out)])
        return carry

    lax.fori_loop(0, nchunk, chunk_body, 0)


def kernel(xyz, grid, xyz_min, xyz_max):
    channels = grid.shape[1]
    shape = xyz.shape[:-1]
    pts = xyz.reshape(-1, 3)
    n = pts.shape[0]
    d0, d1, d2 = grid.shape[2:]
    assert n % (NW * P) == 0

    g = jnp.transpose(grid[0], (1, 2, 3, 0))
    g = jnp.pad(g, ((0, 0), (0, 0), (0, 0), (0, CP - channels)))
    grid_l = g.reshape(d0 * d1 * d2, CP)

    xs = pts[:, 0]
    ys = pts[:, 1]
    zs = pts[:, 2]

    sizes = jnp.array([d0 - 1, d1 - 1, d2 - 1], dtype=jnp.float32)
    scale = sizes / (xyz_max - xyz_min)
    off = -xyz_min * scale
    coef = jnp.concatenate([scale, off, jnp.zeros((2,), jnp.float32)])
    coef = jnp.broadcast_to(coef[:, None], (8, L)).astype(jnp.float32)
    coef = coef.reshape(8 * L)

    body = functools.partial(_tec_body, d0, d1, d2, channels)
    run = pl.kernel(
        body,
        out_type=jax.ShapeDtypeStruct((n * channels,), jnp.float32),
        mesh=plsc.VectorSubcoreMesh(core_axis_name="c", subcore_axis_name="s",
                                    num_cores=NC, num_subcores=NS),
        scratch_types=[
            pltpu.VMEM((8 * L,), jnp.float32),
            pltpu.VMEM((P,), jnp.float32),
            pltpu.VMEM((P,), jnp.float32),
            pltpu.VMEM((P,), jnp.float32),
            pltpu.VMEM((ROWS,), jnp.int32),
            pltpu.VMEM((ROWS,), jnp.float32),
            pltpu.VMEM((ROWS, CP), jnp.float32),
            pltpu.VMEM((P * channels,), jnp.float32),
            pltpu.SemaphoreType.DMA,
        ],
        compiler_params=pltpu.CompilerParams(needs_layout_passes=False,
                                             use_tc_tiling_on_sc=False),
    )
    out = run(grid_l, xs, ys, zs, coef)
    out = out.reshape(*shape, channels)
    if channels == 1:
        out = out.squeeze(-1)
    return out
